# baseline (speedup 1.0000x reference)
import functools

import jax
import jax.numpy as jnp
from jax import lax
from jax.experimental import pallas as pl
from jax.experimental.pallas import tpu as pltpu
from jax.experimental.pallas import tpu_sc as plsc

N = 10000
E = 320000
D = 128
NP = 10240
K = 128
NCH = 2560
EPAD = NCH * K
NSC = 2
NSUB = 16
CPW = NCH // (NSC * NSUB)
RPS = NP // NSUB

_mesh = plsc.VectorSubcoreMesh(core_axis_name="c", subcore_axis_name="s")
_sc_params = pltpu.CompilerParams(use_tc_tiling_on_sc=False)


G = 4
CPW0 = 56
CPW1 = 104
CPWMX = max(CPW0, CPW1)


def _make_agg(F):
    @functools.partial(
        pl.kernel,
        out_type=jax.ShapeDtypeStruct((NSC, NP, F), jnp.float32),
        mesh=_mesh,
        compiler_params=_sc_params,
        scratch_types=[
            pltpu.VMEM_SHARED((NP, F), jnp.float32),
            pltpu.VMEM((CPWMX, K), jnp.int32),
            pltpu.VMEM((CPWMX, K), jnp.int32),
            pltpu.VMEM((G * K, F), jnp.float32),
            pltpu.VMEM((G * K, F), jnp.float32),
            pltpu.SemaphoreType.DMA,
            pltpu.SemaphoreType.DMA,
        ],
    )
    def agg(g_hbm, src_hbm, dst_hbm, zero_hbm, out_hbm, acc,
            sidx, didx, rowsA, rowsB, gsem, ssem):
        cid = lax.axis_index("c")
        sid = lax.axis_index("s")

        pltpu.sync_copy(zero_hbm.at[pl.ds(sid * RPS, RPS)],
                        acc.at[pl.ds(sid * RPS, RPS)])

        def fire_gathers(t, rows):
            return [pltpu.async_copy(g_hbm.at[sidx.at[t * G + b]],
                                     rows.at[pl.ds(b * K, K)], gsem)
                    for b in range(G)]

        def fire_scatters(t, rows):
            return [pltpu.async_copy(rows.at[pl.ds(b * K, K)],
                                     acc.at[didx.at[t * G + b]], ssem,
                                     add=True)
                    for b in range(G)]

        def run(base, cpw):
            pltpu.sync_copy(src_hbm.at[pl.ds(base, cpw)],
                            sidx.at[pl.ds(0, cpw)])
            pltpu.sync_copy(dst_hbm.at[pl.ds(base, cpw)],
                            didx.at[pl.ds(0, cpw)])
            plsc.subcore_barrier()

            @pl.loop(0, cpw // (2 * G))
            def _(j):
                e = 2 * j
                dga = fire_gathers(e, rowsA)
                dgb = fire_gathers(e + 1, rowsB)
                for d in dga:
                    d.wait()
                dsa = fire_scatters(e, rowsA)
                for d in dgb:
                    d.wait()
                dsb = fire_scatters(e + 1, rowsB)
                for d in dsa + dsb:
                    d.wait()

        @pl.when(cid == 0)
        def _():
            run(sid * CPW0, CPW0)

        @pl.when(cid == 1)
        def _():
            run(16 * CPW0 + sid * CPW1, CPW1)

        plsc.subcore_barrier()
        pltpu.sync_copy(acc.at[pl.ds(sid * RPS, RPS)],
                        out_hbm.at[cid, pl.ds(sid * RPS, RPS)])

    return agg


_agg8 = _make_agg(8)


@functools.partial(
    pl.kernel,
    out_type=jax.ShapeDtypeStruct((NSC, NP, 8), jnp.float32),
    mesh=_mesh,
    compiler_params=_sc_params,
    scratch_types=[
        pltpu.VMEM_SHARED((NP, 8), jnp.float32),
        pltpu.VMEM((CPW, K), jnp.int32),
        pltpu.VMEM((K, 8), jnp.float32),
        pltpu.SemaphoreType.DMA,
    ],
)
def _deg(dst_hbm, zero_hbm, ones_hbm, out_hbm, acc, didx, ones_v, sem):
    cid = lax.axis_index("c")
    sid = lax.axis_index("s")
    w = cid * NSUB + sid
    pltpu.sync_copy(dst_hbm.at[pl.ds(w * CPW, CPW)], didx)
    pltpu.sync_copy(zero_hbm.at[pl.ds(sid * RPS, RPS)],
                    acc.at[pl.ds(sid * RPS, RPS)])
    pltpu.sync_copy(ones_hbm, ones_v)
    plsc.subcore_barrier()

    @pl.loop(0, CPW // G)
    def _(t):
        descs = [pltpu.async_copy(ones_v, acc.at[didx.at[t * G + b]], sem,
                                  add=True)
                 for b in range(G)]
        for d in descs:
            d.wait()

    plsc.subcore_barrier()
    pltpu.sync_copy(acc.at[pl.ds(sid * RPS, RPS)],
                    out_hbm.at[cid, pl.ds(sid * RPS, RPS)])


NF = NP * 8 // 128


def _mm_body(x_ref, w_ref, o_ref):
    o_ref[...] = jnp.dot(x_ref[...], w_ref[...],
                         preferred_element_type=jnp.float32)


def _disg_body(dp_ref, y_ref, dis_ref, g_ref):
    deg = dp_ref[0] + dp_ref[1] + 1.0
    dis = lax.rsqrt(deg)
    dis_ref[...] = dis
    g_ref[...] = y_ref[...] * dis


def _post_body(sp_ref, g_ref, dis_ref, w_ref, b_ref, gn_ref):
    s = sp_ref[0] + sp_ref[1] + g_ref[...]
    h = jnp.maximum(s * dis_ref[...] + b_ref[...], 0.0)
    gn_ref[...] = jnp.dot(h, w_ref[...],
                          preferred_element_type=jnp.float32) * dis_ref[...]


def _final_body(sp_ref, g_ref, dis_ref, wc_ref, b3_ref, bc_ref, h_ref, o_ref):
    s = sp_ref[0] + sp_ref[1] + g_ref[...]
    h = jnp.maximum(s * dis_ref[...] + b3_ref[...], 0.0)
    h_ref[...] = h
    o_ref[...] = jnp.dot(h, wc_ref[...],
                         preferred_element_type=jnp.float32) + bc_ref[...]


def kernel(x, edge, W1, b1, W2, b2, W3, b3, Wc, bc):
    f32 = jnp.float32
    eye16 = jnp.eye(16, dtype=f32)
    xp = jnp.pad(x, ((0, NP - N), (0, 0)))
    pad_idx = jnp.full((EPAD - E,), N, jnp.int32)
    src2 = jnp.concatenate([edge[0], pad_idx]).reshape(NCH, K)
    dst2 = jnp.concatenate([edge[1], pad_idx]).reshape(NCH, K)
    z8 = jnp.zeros((NP, 8), f32)
    ones_k = jnp.ones((K, 8), f32)
    W3p = jnp.pad(W3, ((0, 0), (0, 4)))
    Wcp = jnp.pad(Wc, ((0, 4), (0, 0)))
    W2big = jnp.kron(eye16, W2)
    W3big = jnp.kron(eye16, W3p)
    Wcbig = jnp.kron(eye16, Wcp)
    b1f = jnp.tile(b1, 16).reshape(1, 128)
    b2f = jnp.tile(b2, 16).reshape(1, 128)
    b3f = jnp.tile(jnp.pad(b3, (0, 4)), 16).reshape(1, 128)
    bcf = jnp.tile(bc, 16).reshape(1, 256)

    def flat(a):
        return a.reshape(NF, 128)

    degp = _deg(dst2, z8, ones_k)
    y1 = pl.pallas_call(
        _mm_body, out_shape=jax.ShapeDtypeStruct((NP, 8), f32))(xp, W1)
    disf, g1f = pl.pallas_call(
        _disg_body,
        out_shape=[jax.ShapeDtypeStruct((NF, 128), f32),
                   jax.ShapeDtypeStruct((NF, 128), f32)])(
            degp.reshape(NSC, NF, 128), flat(y1))

    s1 = _agg8(g1f.reshape(NP, 8), src2, dst2, z8)
    g2f = pl.pallas_call(
        _post_body, out_shape=jax.ShapeDtypeStruct((NF, 128), f32))(
            s1.reshape(NSC, NF, 128), g1f, disf, W2big, b1f)

    s2 = _agg8(g2f.reshape(NP, 8), src2, dst2, z8)
    g3f = pl.pallas_call(
        _post_body, out_shape=jax.ShapeDtypeStruct((NF, 128), f32))(
            s2.reshape(NSC, NF, 128), g2f, disf, W3big, b2f)

    s3 = _agg8(g3f.reshape(NP, 8), src2, dst2, z8)
    h3f, outf = pl.pallas_call(
        _final_body,
        out_shape=[jax.ShapeDtypeStruct((NF, 128), f32),
                   jax.ShapeDtypeStruct((NF, 256), f32)])(
            s3.reshape(NSC, NF, 128), g3f, disf, Wcbig, b3f, bcf)

    out = outf.reshape(NP, 16)[:N]
    h3 = h3f.reshape(NP, 8)[:N, :4]
    return out, h3


# speedup vs baseline: 1.7959x; 1.7959x over previous
import functools

import jax
import jax.numpy as jnp
from jax import lax
from jax.experimental import pallas as pl
from jax.experimental.pallas import tpu as pltpu
from jax.experimental.pallas import tpu_sc as plsc

N = 10000
E = 320000
D = 128
NP = 10240
K = 128
NCH = 2560
EPAD = NCH * K
NSC = 2
NSUB = 16
CPW = NCH // (NSC * NSUB)
RPS = NP // NSUB

_mesh = plsc.VectorSubcoreMesh(core_axis_name="c", subcore_axis_name="s")
_sc_params = pltpu.CompilerParams(use_tc_tiling_on_sc=False)


G = 4
CPW0 = 80
CPW1 = 80
CPWMX = max(CPW0, CPW1)


def _make_agg(F):
    @functools.partial(
        pl.kernel,
        out_type=jax.ShapeDtypeStruct((NSC, NP, F), jnp.float32),
        mesh=_mesh,
        compiler_params=_sc_params,
        scratch_types=[
            pltpu.VMEM_SHARED((NP, F), jnp.float32),
            pltpu.VMEM_SHARED((NP, F), jnp.float32),
            pltpu.VMEM((CPWMX, K), jnp.int32),
            pltpu.VMEM((CPWMX, K), jnp.int32),
            pltpu.VMEM((G * K, F), jnp.float32),
            pltpu.VMEM((G * K, F), jnp.float32),
            pltpu.SemaphoreType.DMA,
            pltpu.SemaphoreType.DMA,
        ],
    )
    def agg(g_hbm, src_hbm, dst_hbm, zero_hbm, out_hbm, acc, tbl,
            sidx, didx, rowsA, rowsB, gsem, ssem):
        cid = lax.axis_index("c")
        sid = lax.axis_index("s")

        pltpu.sync_copy(zero_hbm.at[pl.ds(sid * RPS, RPS)],
                        acc.at[pl.ds(sid * RPS, RPS)])
        pltpu.sync_copy(g_hbm.at[pl.ds(sid * RPS, RPS)],
                        tbl.at[pl.ds(sid * RPS, RPS)])

        def fire_gathers(t, rows):
            return [pltpu.async_copy(tbl.at[sidx.at[t * G + b]],
                                     rows.at[pl.ds(b * K, K)], gsem)
                    for b in range(G)]

        def fire_scatters(t, rows):
            return [pltpu.async_copy(rows.at[pl.ds(b * K, K)],
                                     acc.at[didx.at[t * G + b]], ssem,
                                     add=True)
                    for b in range(G)]

        def run(base, cpw):
            pltpu.sync_copy(src_hbm.at[pl.ds(base, cpw)],
                            sidx.at[pl.ds(0, cpw)])
            pltpu.sync_copy(dst_hbm.at[pl.ds(base, cpw)],
                            didx.at[pl.ds(0, cpw)])
            plsc.subcore_barrier()

            @pl.loop(0, cpw // (2 * G))
            def _(j):
                e = 2 * j
                dga = fire_gathers(e, rowsA)
                dgb = fire_gathers(e + 1, rowsB)
                for d in dga:
                    d.wait()
                dsa = fire_scatters(e, rowsA)
                for d in dgb:
                    d.wait()
                dsb = fire_scatters(e + 1, rowsB)
                for d in dsa + dsb:
                    d.wait()

        @pl.when(cid == 0)
        def _():
            run(sid * CPW0, CPW0)

        @pl.when(cid == 1)
        def _():
            run(16 * CPW0 + sid * CPW1, CPW1)

        plsc.subcore_barrier()
        pltpu.sync_copy(acc.at[pl.ds(sid * RPS, RPS)],
                        out_hbm.at[cid, pl.ds(sid * RPS, RPS)])

    return agg


_agg8 = _make_agg(8)


@functools.partial(
    pl.kernel,
    out_type=jax.ShapeDtypeStruct((NSC, NP, 8), jnp.float32),
    mesh=_mesh,
    compiler_params=_sc_params,
    scratch_types=[
        pltpu.VMEM_SHARED((NP, 8), jnp.float32),
        pltpu.VMEM((CPW, K), jnp.int32),
        pltpu.VMEM((K, 8), jnp.float32),
        pltpu.SemaphoreType.DMA,
    ],
)
def _deg(dst_hbm, zero_hbm, ones_hbm, out_hbm, acc, didx, ones_v, sem):
    cid = lax.axis_index("c")
    sid = lax.axis_index("s")
    w = cid * NSUB + sid
    pltpu.sync_copy(dst_hbm.at[pl.ds(w * CPW, CPW)], didx)
    pltpu.sync_copy(zero_hbm.at[pl.ds(sid * RPS, RPS)],
                    acc.at[pl.ds(sid * RPS, RPS)])
    pltpu.sync_copy(ones_hbm, ones_v)
    plsc.subcore_barrier()

    @pl.loop(0, CPW // G)
    def _(t):
        descs = [pltpu.async_copy(ones_v, acc.at[didx.at[t * G + b]], sem,
                                  add=True)
                 for b in range(G)]
        for d in descs:
            d.wait()

    plsc.subcore_barrier()
    pltpu.sync_copy(acc.at[pl.ds(sid * RPS, RPS)],
                    out_hbm.at[cid, pl.ds(sid * RPS, RPS)])


NF = NP * 8 // 128


def _mm_body(x_ref, w_ref, o_ref):
    o_ref[...] = jnp.dot(x_ref[...], w_ref[...],
                         preferred_element_type=jnp.float32)


def _disg_body(dp_ref, y_ref, dis_ref, g_ref):
    deg = dp_ref[0] + dp_ref[1] + 1.0
    dis = lax.rsqrt(deg)
    dis_ref[...] = dis
    g_ref[...] = y_ref[...] * dis


def _post_body(sp_ref, g_ref, dis_ref, w_ref, b_ref, gn_ref):
    s = sp_ref[0] + sp_ref[1] + g_ref[...]
    h = jnp.maximum(s * dis_ref[...] + b_ref[...], 0.0)
    gn_ref[...] = jnp.dot(h, w_ref[...],
                          preferred_element_type=jnp.float32) * dis_ref[...]


def _final_body(sp_ref, g_ref, dis_ref, wc_ref, b3_ref, bc_ref, h_ref, o_ref):
    s = sp_ref[0] + sp_ref[1] + g_ref[...]
    h = jnp.maximum(s * dis_ref[...] + b3_ref[...], 0.0)
    h_ref[...] = h
    o_ref[...] = jnp.dot(h, wc_ref[...],
                         preferred_element_type=jnp.float32) + bc_ref[...]


def kernel(x, edge, W1, b1, W2, b2, W3, b3, Wc, bc):
    f32 = jnp.float32
    eye16 = jnp.eye(16, dtype=f32)
    xp = jnp.pad(x, ((0, NP - N), (0, 0)))
    pad_idx = jnp.full((EPAD - E,), N, jnp.int32)
    src2 = jnp.concatenate([edge[0], pad_idx]).reshape(NCH, K)
    dst2 = jnp.concatenate([edge[1], pad_idx]).reshape(NCH, K)
    z8 = jnp.zeros((NP, 8), f32)
    ones_k = jnp.ones((K, 8), f32)
    W3p = jnp.pad(W3, ((0, 0), (0, 4)))
    Wcp = jnp.pad(Wc, ((0, 4), (0, 0)))
    W2big = jnp.kron(eye16, W2)
    W3big = jnp.kron(eye16, W3p)
    Wcbig = jnp.kron(eye16, Wcp)
    b1f = jnp.tile(b1, 16).reshape(1, 128)
    b2f = jnp.tile(b2, 16).reshape(1, 128)
    b3f = jnp.tile(jnp.pad(b3, (0, 4)), 16).reshape(1, 128)
    bcf = jnp.tile(bc, 16).reshape(1, 256)

    def flat(a):
        return a.reshape(NF, 128)

    degp = _deg(dst2, z8, ones_k)
    y1 = pl.pallas_call(
        _mm_body, out_shape=jax.ShapeDtypeStruct((NP, 8), f32))(xp, W1)
    disf, g1f = pl.pallas_call(
        _disg_body,
        out_shape=[jax.ShapeDtypeStruct((NF, 128), f32),
                   jax.ShapeDtypeStruct((NF, 128), f32)])(
            degp.reshape(NSC, NF, 128), flat(y1))

    s1 = _agg8(g1f.reshape(NP, 8), src2, dst2, z8)
    g2f = pl.pallas_call(
        _post_body, out_shape=jax.ShapeDtypeStruct((NF, 128), f32))(
            s1.reshape(NSC, NF, 128), g1f, disf, W2big, b1f)

    s2 = _agg8(g2f.reshape(NP, 8), src2, dst2, z8)
    g3f = pl.pallas_call(
        _post_body, out_shape=jax.ShapeDtypeStruct((NF, 128), f32))(
            s2.reshape(NSC, NF, 128), g2f, disf, W3big, b2f)

    s3 = _agg8(g3f.reshape(NP, 8), src2, dst2, z8)
    h3f, outf = pl.pallas_call(
        _final_body,
        out_shape=[jax.ShapeDtypeStruct((NF, 128), f32),
                   jax.ShapeDtypeStruct((NF, 256), f32)])(
            s3.reshape(NSC, NF, 128), g3f, disf, Wcbig, b3f, bcf)

    out = outf.reshape(NP, 16)[:N]
    h3 = h3f.reshape(NP, 8)[:N, :4]
    return out, h3


# speedup vs baseline: 1.8157x; 1.0110x over previous
import functools

import jax
import jax.numpy as jnp
from jax import lax
from jax.experimental import pallas as pl
from jax.experimental.pallas import tpu as pltpu
from jax.experimental.pallas import tpu_sc as plsc

N = 10000
E = 320000
D = 128
NP = 10240
K = 128
NCH = 2560
EPAD = NCH * K
NSC = 2
NSUB = 16
CPW = NCH // (NSC * NSUB)
RPS = NP // NSUB

_mesh = plsc.VectorSubcoreMesh(core_axis_name="c", subcore_axis_name="s")
_sc_params = pltpu.CompilerParams(use_tc_tiling_on_sc=False)


G = 4
CPW0 = 88
CPW1 = 72
CPWMX = max(CPW0, CPW1)


def _make_agg(F):
    @functools.partial(
        pl.kernel,
        out_type=jax.ShapeDtypeStruct((NSC, NP, F), jnp.float32),
        mesh=_mesh,
        compiler_params=_sc_params,
        scratch_types=[
            pltpu.VMEM_SHARED((NP, F), jnp.float32),
            pltpu.VMEM_SHARED((NP, F), jnp.float32),
            pltpu.VMEM((CPWMX, K), jnp.int32),
            pltpu.VMEM((CPWMX, K), jnp.int32),
            pltpu.VMEM((G * K, F), jnp.float32),
            pltpu.VMEM((G * K, F), jnp.float32),
            pltpu.SemaphoreType.DMA,
            pltpu.SemaphoreType.DMA,
        ],
    )
    def agg(g_hbm, src_hbm, dst_hbm, zero_hbm, out_hbm, acc, tbl,
            sidx, didx, rowsA, rowsB, gsem, ssem):
        cid = lax.axis_index("c")
        sid = lax.axis_index("s")

        pltpu.sync_copy(zero_hbm.at[pl.ds(sid * RPS, RPS)],
                        acc.at[pl.ds(sid * RPS, RPS)])
        pltpu.sync_copy(g_hbm.at[pl.ds(sid * RPS, RPS)],
                        tbl.at[pl.ds(sid * RPS, RPS)])

        def fire_gathers(t, rows):
            return [pltpu.async_copy(tbl.at[sidx.at[t * G + b]],
                                     rows.at[pl.ds(b * K, K)], gsem)
                    for b in range(G)]

        def fire_scatters(t, rows):
            return [pltpu.async_copy(rows.at[pl.ds(b * K, K)],
                                     acc.at[didx.at[t * G + b]], ssem,
                                     add=True)
                    for b in range(G)]

        def run(base, cpw):
            pltpu.sync_copy(src_hbm.at[pl.ds(base, cpw)],
                            sidx.at[pl.ds(0, cpw)])
            pltpu.sync_copy(dst_hbm.at[pl.ds(base, cpw)],
                            didx.at[pl.ds(0, cpw)])
            plsc.subcore_barrier()

            @pl.loop(0, cpw // (2 * G))
            def _(j):
                e = 2 * j
                dga = fire_gathers(e, rowsA)
                dgb = fire_gathers(e + 1, rowsB)
                for d in dga:
                    d.wait()
                dsa = fire_scatters(e, rowsA)
                for d in dgb:
                    d.wait()
                dsb = fire_scatters(e + 1, rowsB)
                for d in dsa + dsb:
                    d.wait()

        @pl.when(cid == 0)
        def _():
            run(sid * CPW0, CPW0)

        @pl.when(cid == 1)
        def _():
            run(16 * CPW0 + sid * CPW1, CPW1)

        plsc.subcore_barrier()
        pltpu.sync_copy(acc.at[pl.ds(sid * RPS, RPS)],
                        out_hbm.at[cid, pl.ds(sid * RPS, RPS)])

    return agg


_agg8 = _make_agg(8)


@functools.partial(
    pl.kernel,
    out_type=jax.ShapeDtypeStruct((NSC, NP, 8), jnp.float32),
    mesh=_mesh,
    compiler_params=_sc_params,
    scratch_types=[
        pltpu.VMEM_SHARED((NP, 8), jnp.float32),
        pltpu.VMEM((CPW, K), jnp.int32),
        pltpu.VMEM((K, 8), jnp.float32),
        pltpu.SemaphoreType.DMA,
    ],
)
def _deg(dst_hbm, zero_hbm, ones_hbm, out_hbm, acc, didx, ones_v, sem):
    cid = lax.axis_index("c")
    sid = lax.axis_index("s")
    w = cid * NSUB + sid
    pltpu.sync_copy(dst_hbm.at[pl.ds(w * CPW, CPW)], didx)
    pltpu.sync_copy(zero_hbm.at[pl.ds(sid * RPS, RPS)],
                    acc.at[pl.ds(sid * RPS, RPS)])
    pltpu.sync_copy(ones_hbm, ones_v)
    plsc.subcore_barrier()

    @pl.loop(0, CPW // G)
    def _(t):
        descs = [pltpu.async_copy(ones_v, acc.at[didx.at[t * G + b]], sem,
                                  add=True)
                 for b in range(G)]
        for d in descs:
            d.wait()

    plsc.subcore_barrier()
    pltpu.sync_copy(acc.at[pl.ds(sid * RPS, RPS)],
                    out_hbm.at[cid, pl.ds(sid * RPS, RPS)])


NF = NP * 8 // 128


def _mm_body(x_ref, w_ref, o_ref):
    o_ref[...] = jnp.dot(x_ref[...], w_ref[...],
                         preferred_element_type=jnp.float32)


def _disg_body(dp_ref, y_ref, dis_ref, g_ref):
    deg = dp_ref[0] + dp_ref[1] + 1.0
    dis = lax.rsqrt(deg)
    dis_ref[...] = dis
    g_ref[...] = y_ref[...] * dis


def _post_body(sp_ref, g_ref, dis_ref, w_ref, b_ref, gn_ref):
    s = sp_ref[0] + sp_ref[1] + g_ref[...]
    h = jnp.maximum(s * dis_ref[...] + b_ref[...], 0.0)
    gn_ref[...] = jnp.dot(h, w_ref[...],
                          preferred_element_type=jnp.float32) * dis_ref[...]


def _final_body(sp_ref, g_ref, dis_ref, wc_ref, b3_ref, bc_ref, h_ref, o_ref):
    s = sp_ref[0] + sp_ref[1] + g_ref[...]
    h = jnp.maximum(s * dis_ref[...] + b3_ref[...], 0.0)
    h_ref[...] = h
    o_ref[...] = jnp.dot(h, wc_ref[...],
                         preferred_element_type=jnp.float32) + bc_ref[...]


def kernel(x, edge, W1, b1, W2, b2, W3, b3, Wc, bc):
    f32 = jnp.float32
    eye16 = jnp.eye(16, dtype=f32)
    xp = jnp.pad(x, ((0, NP - N), (0, 0)))
    pad_idx = jnp.full((EPAD - E,), N, jnp.int32)
    src2 = jnp.concatenate([edge[0], pad_idx]).reshape(NCH, K)
    dst2 = jnp.concatenate([edge[1], pad_idx]).reshape(NCH, K)
    z8 = jnp.zeros((NP, 8), f32)
    ones_k = jnp.ones((K, 8), f32)
    W3p = jnp.pad(W3, ((0, 0), (0, 4)))
    Wcp = jnp.pad(Wc, ((0, 4), (0, 0)))
    W2big = jnp.kron(eye16, W2)
    W3big = jnp.kron(eye16, W3p)
    Wcbig = jnp.kron(eye16, Wcp)
    b1f = jnp.tile(b1, 16).reshape(1, 128)
    b2f = jnp.tile(b2, 16).reshape(1, 128)
    b3f = jnp.tile(jnp.pad(b3, (0, 4)), 16).reshape(1, 128)
    bcf = jnp.tile(bc, 16).reshape(1, 256)

    def flat(a):
        return a.reshape(NF, 128)

    degp = _deg(dst2, z8, ones_k)
    y1 = pl.pallas_call(
        _mm_body, out_shape=jax.ShapeDtypeStruct((NP, 8), f32))(xp, W1)
    disf, g1f = pl.pallas_call(
        _disg_body,
        out_shape=[jax.ShapeDtypeStruct((NF, 128), f32),
                   jax.ShapeDtypeStruct((NF, 128), f32)])(
            degp.reshape(NSC, NF, 128), flat(y1))

    s1 = _agg8(g1f.reshape(NP, 8), src2, dst2, z8)
    g2f = pl.pallas_call(
        _post_body, out_shape=jax.ShapeDtypeStruct((NF, 128), f32))(
            s1.reshape(NSC, NF, 128), g1f, disf, W2big, b1f)

    s2 = _agg8(g2f.reshape(NP, 8), src2, dst2, z8)
    g3f = pl.pallas_call(
        _post_body, out_shape=jax.ShapeDtypeStruct((NF, 128), f32))(
            s2.reshape(NSC, NF, 128), g2f, disf, W3big, b2f)

    s3 = _agg8(g3f.reshape(NP, 8), src2, dst2, z8)
    h3f, outf = pl.pallas_call(
        _final_body,
        out_shape=[jax.ShapeDtypeStruct((NF, 128), f32),
                   jax.ShapeDtypeStruct((NF, 256), f32)])(
            s3.reshape(NSC, NF, 128), g3f, disf, Wcbig, b3f, bcf)

    out = outf.reshape(NP, 16)[:N]
    h3 = h3f.reshape(NP, 8)[:N, :4]
    return out, h3


# speedup vs baseline: 1.8500x; 1.0189x over previous
import functools

import jax
import jax.numpy as jnp
from jax import lax
from jax.experimental import pallas as pl
from jax.experimental.pallas import tpu as pltpu
from jax.experimental.pallas import tpu_sc as plsc

N = 10000
E = 320000
D = 128
NP = 10240
K = 128
NCH = 2560
EPAD = NCH * K
NSC = 2
NSUB = 16
CPW = NCH // (NSC * NSUB)
RPS = NP // NSUB

_mesh = plsc.VectorSubcoreMesh(core_axis_name="c", subcore_axis_name="s")
_sc_params = pltpu.CompilerParams(use_tc_tiling_on_sc=False)


G = 2
NBUF = 4
CPW0 = 88
CPW1 = 72
CPWMX = max(CPW0, CPW1)


def _make_agg(F):
    @functools.partial(
        pl.kernel,
        out_type=jax.ShapeDtypeStruct((NSC, NP, F), jnp.float32),
        mesh=_mesh,
        compiler_params=_sc_params,
        scratch_types=[
            pltpu.VMEM_SHARED((NP, F), jnp.float32),
            pltpu.VMEM_SHARED((NP, F), jnp.float32),
            pltpu.VMEM((CPWMX, K), jnp.int32),
            pltpu.VMEM((CPWMX, K), jnp.int32),
        ] + [pltpu.VMEM((G * K, F), jnp.float32)
             for _ in range(NBUF)] + [
            pltpu.SemaphoreType.DMA,
            pltpu.SemaphoreType.DMA,
        ],
    )
    def agg(g_hbm, src_hbm, dst_hbm, zero_hbm, out_hbm, acc, tbl,
            sidx, didx, *bufs_and_sems):
        bufs = bufs_and_sems[:NBUF]
        gsem, ssem = bufs_and_sems[NBUF:]
        cid = lax.axis_index("c")
        sid = lax.axis_index("s")

        pltpu.sync_copy(zero_hbm.at[pl.ds(sid * RPS, RPS)],
                        acc.at[pl.ds(sid * RPS, RPS)])
        pltpu.sync_copy(g_hbm.at[pl.ds(sid * RPS, RPS)],
                        tbl.at[pl.ds(sid * RPS, RPS)])

        def fire_gathers(t, rows):
            return [pltpu.async_copy(tbl.at[sidx.at[t * G + b]],
                                     rows.at[pl.ds(b * K, K)], gsem)
                    for b in range(G)]

        def fire_scatters(t, rows):
            return [pltpu.async_copy(rows.at[pl.ds(b * K, K)],
                                     acc.at[didx.at[t * G + b]], ssem,
                                     add=True)
                    for b in range(G)]

        def run(base, cpw):
            pltpu.sync_copy(src_hbm.at[pl.ds(base, cpw)],
                            sidx.at[pl.ds(0, cpw)])
            pltpu.sync_copy(dst_hbm.at[pl.ds(base, cpw)],
                            didx.at[pl.ds(0, cpw)])
            plsc.subcore_barrier()

            @pl.loop(0, cpw // (NBUF * G))
            def _(j):
                e = NBUF * j
                dg = [fire_gathers(e + i, bufs[i]) for i in range(NBUF)]
                ds = []
                for i in range(NBUF):
                    for d in dg[i]:
                        d.wait()
                    ds += fire_scatters(e + i, bufs[i])
                for d in ds:
                    d.wait()

        @pl.when(cid == 0)
        def _():
            run(sid * CPW0, CPW0)

        @pl.when(cid == 1)
        def _():
            run(16 * CPW0 + sid * CPW1, CPW1)

        plsc.subcore_barrier()
        pltpu.sync_copy(acc.at[pl.ds(sid * RPS, RPS)],
                        out_hbm.at[cid, pl.ds(sid * RPS, RPS)])

    return agg


_agg8 = _make_agg(8)


@functools.partial(
    pl.kernel,
    out_type=jax.ShapeDtypeStruct((NSC, NP, 8), jnp.float32),
    mesh=_mesh,
    compiler_params=_sc_params,
    scratch_types=[
        pltpu.VMEM_SHARED((NP, 8), jnp.float32),
        pltpu.VMEM((CPW, K), jnp.int32),
        pltpu.VMEM((K, 8), jnp.float32),
        pltpu.SemaphoreType.DMA,
    ],
)
def _deg(dst_hbm, zero_hbm, ones_hbm, out_hbm, acc, didx, ones_v, sem):
    cid = lax.axis_index("c")
    sid = lax.axis_index("s")
    w = cid * NSUB + sid
    pltpu.sync_copy(dst_hbm.at[pl.ds(w * CPW, CPW)], didx)
    pltpu.sync_copy(zero_hbm.at[pl.ds(sid * RPS, RPS)],
                    acc.at[pl.ds(sid * RPS, RPS)])
    pltpu.sync_copy(ones_hbm, ones_v)
    plsc.subcore_barrier()

    @pl.loop(0, CPW // G)
    def _(t):
        descs = [pltpu.async_copy(ones_v, acc.at[didx.at[t * G + b]], sem,
                                  add=True)
                 for b in range(G)]
        for d in descs:
            d.wait()

    plsc.subcore_barrier()
    pltpu.sync_copy(acc.at[pl.ds(sid * RPS, RPS)],
                    out_hbm.at[cid, pl.ds(sid * RPS, RPS)])


NF = NP * 8 // 128


def _mm_body(x_ref, w_ref, o_ref):
    o_ref[...] = jnp.dot(x_ref[...], w_ref[...],
                         preferred_element_type=jnp.float32)


def _disg_body(dp_ref, y_ref, dis_ref, g_ref):
    deg = dp_ref[0] + dp_ref[1] + 1.0
    dis = lax.rsqrt(deg)
    dis_ref[...] = dis
    g_ref[...] = y_ref[...] * dis


def _post_body(sp_ref, g_ref, dis_ref, w_ref, b_ref, gn_ref):
    s = sp_ref[0] + sp_ref[1] + g_ref[...]
    h = jnp.maximum(s * dis_ref[...] + b_ref[...], 0.0)
    gn_ref[...] = jnp.dot(h, w_ref[...],
                          preferred_element_type=jnp.float32) * dis_ref[...]


def _final_body(sp_ref, g_ref, dis_ref, wc_ref, b3_ref, bc_ref, h_ref, o_ref):
    s = sp_ref[0] + sp_ref[1] + g_ref[...]
    h = jnp.maximum(s * dis_ref[...] + b3_ref[...], 0.0)
    h_ref[...] = h
    o_ref[...] = jnp.dot(h, wc_ref[...],
                         preferred_element_type=jnp.float32) + bc_ref[...]


def kernel(x, edge, W1, b1, W2, b2, W3, b3, Wc, bc):
    f32 = jnp.float32
    eye16 = jnp.eye(16, dtype=f32)
    xp = jnp.pad(x, ((0, NP - N), (0, 0)))
    pad_idx = jnp.full((EPAD - E,), N, jnp.int32)
    src2 = jnp.concatenate([edge[0], pad_idx]).reshape(NCH, K)
    dst2 = jnp.concatenate([edge[1], pad_idx]).reshape(NCH, K)
    z8 = jnp.zeros((NP, 8), f32)
    ones_k = jnp.ones((K, 8), f32)
    W3p = jnp.pad(W3, ((0, 0), (0, 4)))
    Wcp = jnp.pad(Wc, ((0, 4), (0, 0)))
    W2big = jnp.kron(eye16, W2)
    W3big = jnp.kron(eye16, W3p)
    Wcbig = jnp.kron(eye16, Wcp)
    b1f = jnp.tile(b1, 16).reshape(1, 128)
    b2f = jnp.tile(b2, 16).reshape(1, 128)
    b3f = jnp.tile(jnp.pad(b3, (0, 4)), 16).reshape(1, 128)
    bcf = jnp.tile(bc, 16).reshape(1, 256)

    def flat(a):
        return a.reshape(NF, 128)

    degp = _deg(dst2, z8, ones_k)
    y1 = pl.pallas_call(
        _mm_body, out_shape=jax.ShapeDtypeStruct((NP, 8), f32))(xp, W1)
    disf, g1f = pl.pallas_call(
        _disg_body,
        out_shape=[jax.ShapeDtypeStruct((NF, 128), f32),
                   jax.ShapeDtypeStruct((NF, 128), f32)])(
            degp.reshape(NSC, NF, 128), flat(y1))

    s1 = _agg8(g1f.reshape(NP, 8), src2, dst2, z8)
    g2f = pl.pallas_call(
        _post_body, out_shape=jax.ShapeDtypeStruct((NF, 128), f32))(
            s1.reshape(NSC, NF, 128), g1f, disf, W2big, b1f)

    s2 = _agg8(g2f.reshape(NP, 8), src2, dst2, z8)
    g3f = pl.pallas_call(
        _post_body, out_shape=jax.ShapeDtypeStruct((NF, 128), f32))(
            s2.reshape(NSC, NF, 128), g2f, disf, W3big, b2f)

    s3 = _agg8(g3f.reshape(NP, 8), src2, dst2, z8)
    h3f, outf = pl.pallas_call(
        _final_body,
        out_shape=[jax.ShapeDtypeStruct((NF, 128), f32),
                   jax.ShapeDtypeStruct((NF, 256), f32)])(
            s3.reshape(NSC, NF, 128), g3f, disf, Wcbig, b3f, bcf)

    out = outf.reshape(NP, 16)[:N]
    h3 = h3f.reshape(NP, 8)[:N, :4]
    return out, h3
